# baseline (speedup 1.0000x reference)
import jax
import jax.numpy as jnp
from jax.experimental import pallas as pl
from jax.experimental.pallas import tpu as pltpu

_B, _N, _D, _H = 8, 256, 4, 32
_STEPS = 8
_GRP = 4
_NG = _B // _GRP
_HP = _H * _GRP
_SC = 32
_NCH = _N // _SC


def _fused_kernel(x_ref, M_ref, W0blk_ref, eeb0t_ref, W1blk_ref, eeb1t_ref,
                  neW0blk_ref, neb0t_ref, neW1blk_ref, neb1t_ref,
                  ndW0blk_ref, ndb0t_ref, ndW1blk_ref, ndb1t_ref,
                  outWblk_ref, outbt_ref, out_ref):
    M = M_ref[:]
    W0blk = W0blk_ref[:]
    eeb0t = eeb0t_ref[:]
    W1blk = W1blk_ref[:]
    eeb1t = eeb1t_ref[:]
    neW0blk = neW0blk_ref[:]; neb0t = neb0t_ref[:]
    neW1blk = neW1blk_ref[:]; neb1t = neb1t_ref[:]
    ndW0blk = ndW0blk_ref[:]
    ndb0t = ndb0t_ref[:]
    ndW1blk = ndW1blk_ref[:]; ndb1t = ndb1t_ref[:]
    outWblk = outWblk_ref[:]
    outbt = outbt_ref[:]

    def dot(a, b):
        return jax.lax.dot_general(a, b, (((1,), (0,)), ((), ())),
                                   preferred_element_type=jnp.float32)

    zc = jnp.zeros((_N, _D), jnp.float32)

    def step_fn(i, xb):
        xs = jnp.concatenate(
            [v for b in range(_GRP) for v in (xb[b], zc)], axis=1)
        xt = jnp.concatenate(
            [v for b in range(_GRP) for v in (zc, xb[b])], axis=1)

        acc = jnp.zeros((_N, _HP), jnp.float32)
        for c in range(_NCH):
            xsc = xs[c * _SC:(c + 1) * _SC, :]
            a = xsc[:, None, :] + xt[None, :, :]
            h1 = jnp.maximum(
                dot(a.reshape(_SC * _N, 2 * _D * _GRP), W0blk) + eeb0t, 0.0)
            h2 = jnp.maximum(dot(h1, W1blk) + eeb1t, 0.0)
            h2 = h2.reshape(_SC, _N, _HP)
            Mc = M[c * _SC:(c + 1) * _SC, :]
            for s in range(_SC):
                acc = acc + h2[s] * Mc[s, :, None]
        nm = jnp.maximum(dot(acc, neW0blk) + neb0t, 0.0)
        nm = jnp.maximum(dot(nm, neW1blk) + neb1t, 0.0)
        d2 = jnp.concatenate(
            [v for b in range(_GRP)
             for v in (xb[b], nm[:, b * _H:(b + 1) * _H])], axis=1)
        h = jnp.maximum(dot(d2, ndW0blk) + ndb0t, 0.0)
        h = jnp.maximum(dot(h, ndW1blk) + ndb1t, 0.0)
        xpk = jnp.concatenate(xb, axis=1)
        nxt = dot(h, outWblk) + outbt + xpk
        out_ref[0, i] = nxt
        return tuple(nxt[:, b * _D:(b + 1) * _D] for b in range(_GRP))

    x0 = tuple(x_ref[0, b * _N:(b + 1) * _N, :] for b in range(_GRP))
    jax.lax.fori_loop(0, _STEPS, step_fn, x0)


def kernel(time_segs, ee_W0, ee_b0, ee_W1, ee_b1, ne_W0, ne_b0, ne_W1, ne_b1,
           nd_W0, nd_b0, nd_W1, nd_b1, out_W, out_b):
    x0 = time_segs.reshape(_NG, _GRP * _N, _D)
    eye = jnp.eye(_GRP, dtype=jnp.float32)
    M = jnp.ones((_N, _N), jnp.float32) - jnp.eye(_N, dtype=jnp.float32)
    full = lambda shape: pl.BlockSpec(shape, lambda g: (0,) * len(shape))
    out = pl.pallas_call(
        _fused_kernel,
        grid=(_NG,),
        in_specs=[
            pl.BlockSpec((1, _GRP * _N, _D), lambda g: (g, 0, 0)),
            full((_N, _N)),
            full((2 * _D * _GRP, _HP)), full((1, _HP)),
            full((_HP, _HP)), full((1, _HP)),
            full((_HP, _HP)), full((1, _HP)),
            full((_HP, _HP)), full((1, _HP)),
            full(((_D + _H) * _GRP, _HP)), full((1, _HP)),
            full((_HP, _HP)), full((1, _HP)),
            full((_HP, _GRP * _D)), full((1, _GRP * _D)),
        ],
        out_specs=pl.BlockSpec((1, _STEPS, _N, _GRP * _D),
                               lambda g: (g, 0, 0, 0)),
        out_shape=jax.ShapeDtypeStruct((_NG, _STEPS, _N, _GRP * _D),
                                       jnp.float32),
        compiler_params=pltpu.CompilerParams(
            dimension_semantics=("parallel",)),
    )(
        x0, M,
        jnp.kron(eye, ee_W0), jnp.tile(ee_b0, _GRP)[None, :],
        jnp.kron(eye, ee_W1), jnp.tile(ee_b1, _GRP)[None, :],
        jnp.kron(eye, ne_W0), jnp.tile(ne_b0, _GRP)[None, :],
        jnp.kron(eye, ne_W1), jnp.tile(ne_b1, _GRP)[None, :],
        jnp.kron(eye, nd_W0), jnp.tile(nd_b0, _GRP)[None, :],
        jnp.kron(eye, nd_W1), jnp.tile(nd_b1, _GRP)[None, :],
        jnp.kron(eye, out_W), jnp.tile(out_b, _GRP)[None, :],
    )
    out = out.reshape(_NG, _STEPS, _N, _GRP, _D)
    out = out.transpose(0, 3, 1, 2, 4).reshape(_B, _STEPS, _N, _D)
    return out


# speedup vs baseline: 1.1100x; 1.1100x over previous
import jax
import jax.numpy as jnp
from jax.experimental import pallas as pl
from jax.experimental.pallas import tpu as pltpu

_B, _N, _D, _H = 8, 256, 4, 32
_STEPS = 8
_GRP = 4
_NG = _B // _GRP
_HP = _H * _GRP
_SC = 32
_NCH = _N // _SC


def _fused_kernel(x_ref, M_ref, W0blk_ref, eeb0t_ref, W1blk_ref, eeb1t_ref,
                  neW0blk_ref, neb0t_ref, neW1blk_ref, neb1t_ref,
                  ndW0blk_ref, ndb0t_ref, ndW1blk_ref, ndb1t_ref,
                  outWblk_ref, outbt_ref, out_ref):
    M = M_ref[:]
    W0blk = (W0blk_ref[0:2 * _D * _B, :],
             W0blk_ref[2 * _D * _B:4 * _D * _B, :])
    eeb0t = eeb0t_ref[:]
    W1blk = W1blk_ref[:]
    eeb1t = eeb1t_ref[:]
    neW0blk = neW0blk_ref[:]; neb0t = neb0t_ref[:]
    neW1blk = neW1blk_ref[:]; neb1t = neb1t_ref[:]
    ndW0blk = ndW0blk_ref[:]
    ndb0t = ndb0t_ref[:]
    ndW1blk = ndW1blk_ref[:]; ndb1t = ndb1t_ref[:]
    outWblk = outWblk_ref[:]
    outbt = outbt_ref[:]

    def dot(a, b):
        return jax.lax.dot_general(a, b, (((1,), (0,)), ((), ())),
                                   preferred_element_type=jnp.float32)

    zc = jnp.zeros((_N, _D), jnp.float32)

    def step_fn(i, xs_all):
        xs = jnp.concatenate(
            [v for b in range(_B) for v in (xs_all[b], zc)], axis=1)
        xt = jnp.concatenate(
            [v for b in range(_B) for v in (zc, xs_all[b])], axis=1)

        accs = [jnp.zeros((_N, _HP), jnp.float32) for _ in range(_NG)]
        for c in range(_NCH):
            xsc = xs[c * _SC:(c + 1) * _SC, :]
            a = (xsc[:, None, :] + xt[None, :, :]).reshape(
                _SC * _N, 2 * _D * _B)
            Mc = M[c * _SC:(c + 1) * _SC, :]
            for grp in range(_NG):
                h1 = jnp.maximum(dot(a, W0blk[grp]) + eeb0t, 0.0)
                h2 = jnp.maximum(dot(h1, W1blk) + eeb1t, 0.0)
                h2 = h2.reshape(_SC, _N, _HP)
                acc = accs[grp]
                for s in range(_SC):
                    acc = acc + h2[s] * Mc[s, :, None]
                accs[grp] = acc

        new_states = []
        for grp in range(_NG):
            xb = xs_all[grp * _GRP:(grp + 1) * _GRP]
            acc = accs[grp]
            nm = jnp.maximum(dot(acc, neW0blk) + neb0t, 0.0)
            nm = jnp.maximum(dot(nm, neW1blk) + neb1t, 0.0)
            d2 = jnp.concatenate(
                [v for b in range(_GRP)
                 for v in (xb[b], nm[:, b * _H:(b + 1) * _H])], axis=1)
            h = jnp.maximum(dot(d2, ndW0blk) + ndb0t, 0.0)
            h = jnp.maximum(dot(h, ndW1blk) + ndb1t, 0.0)
            xpk = jnp.concatenate(xb, axis=1)
            nxt = dot(h, outWblk) + outbt + xpk
            out_ref[i, grp] = nxt
            for b in range(_GRP):
                new_states.append(nxt[:, b * _D:(b + 1) * _D])
        return tuple(new_states)

    x0 = tuple(x_ref[b * _N:(b + 1) * _N, :] for b in range(_B))
    jax.lax.fori_loop(0, _STEPS, step_fn, x0)


def kernel(time_segs, ee_W0, ee_b0, ee_W1, ee_b1, ne_W0, ne_b0, ne_W1, ne_b1,
           nd_W0, nd_b0, nd_W1, nd_b1, out_W, out_b):
    x0 = time_segs.reshape(_B * _N, _D)
    eye = jnp.eye(_GRP, dtype=jnp.float32)
    M = jnp.ones((_N, _N), jnp.float32) - jnp.eye(_N, dtype=jnp.float32)
    W0g0 = jnp.concatenate(
        [jnp.kron(eye, ee_W0), jnp.zeros((2 * _D * _GRP, _HP), jnp.float32)], axis=0)
    W0g1 = jnp.concatenate(
        [jnp.zeros((2 * _D * _GRP, _HP), jnp.float32), jnp.kron(eye, ee_W0)], axis=0)
    W0stk = jnp.concatenate([W0g0, W0g1], axis=0)
    out = pl.pallas_call(
        _fused_kernel,
        out_shape=jax.ShapeDtypeStruct((_STEPS, _NG, _N, _GRP * _D), jnp.float32),
    )(
        x0, M,
        W0stk, jnp.tile(ee_b0, _GRP)[None, :],
        jnp.kron(eye, ee_W1), jnp.tile(ee_b1, _GRP)[None, :],
        jnp.kron(eye, ne_W0), jnp.tile(ne_b0, _GRP)[None, :],
        jnp.kron(eye, ne_W1), jnp.tile(ne_b1, _GRP)[None, :],
        jnp.kron(eye, nd_W0), jnp.tile(nd_b0, _GRP)[None, :],
        jnp.kron(eye, nd_W1), jnp.tile(nd_b1, _GRP)[None, :],
        jnp.kron(eye, out_W), jnp.tile(out_b, _GRP)[None, :],
    )
    out = out.reshape(_STEPS, _NG, _N, _GRP, _D)
    out = out.transpose(1, 3, 0, 2, 4).reshape(_B, _STEPS, _N, _D)
    return out
